# baseline (speedup 1.0000x reference)
import functools

import jax
import jax.numpy as jnp
from jax.experimental import pallas as pl
from jax.experimental.pallas import tpu as pltpu

B, L, D = 4, 128, 768
SPAN_PRUNING = 0.3
BLK = 32
NBLK = (B * L) // BLK
N_ITER = 31


def _proj_body(t_ref, w_ref, b_ref, s_ref, e_ref):
    x = t_ref[...]
    x2 = x.reshape(BLK * L, D)
    r = jnp.dot(x2, w_ref[...], preferred_element_type=jnp.float32)
    r = r + b_ref[...]
    s_ref[...] = r[:, 0].reshape(BLK, L)
    e_ref[...] = r[:, 1].reshape(BLK, L)


def _bce_elem(logits, targets):
    return (jnp.maximum(logits, 0.0) - logits * targets
            + jnp.log1p(jnp.exp(-jnp.abs(logits))))


def _head_body(ls_ref, le_ref, labs_ref, labe_ref, ia_ref, labias_ref,
               labiae_ref, am_ref, wia_ref, bia_ref,
               loss_s_ref, loss_e_ref, loss_ias_ref, loss_iae_ref,
               ms_ref, me_ref, mias_ref, miae_ref):
    am = am_ref[...]
    msum = jnp.sum(am, axis=1, keepdims=True)
    ml = msum - 3.0
    ln = (ml * SPAN_PRUNING).astype(jnp.int32)
    ln = jnp.maximum(ln, 10)
    maxl = (ml * ml).astype(jnp.int32)
    k = jnp.minimum(ln, maxl)

    def table_pred(l_ref, lab_ref, loss_ref):
        logits = l_ref[...]
        lab = lab_ref[...]
        w = jnp.where(lab >= 0, 1.0, 0.0)
        elem = _bce_elem(logits, lab.astype(jnp.float32))
        loss_ref[...] = jnp.sum(jnp.sum(w * elem, axis=0)).reshape(1, 1) \
            / float(B * L * L)
        p = jax.nn.sigmoid(logits) * w
        return jax.lax.bitcast_convert_type(p.reshape(B, L, L), jnp.int32)

    pbs = table_pred(ls_ref, labs_ref, loss_s_ref)
    pbe = table_pred(le_ref, labe_ref, loss_e_ref)

    x16 = ia_ref[...].astype(jnp.bfloat16).astype(jnp.float32)
    wia = wia_ref[...]
    bia = bia_ref[...]

    def ia_pred(col, lab_ref, loss_ref):
        wvec = wia[0, col * D:(col + 1) * D].reshape(1, 1, D)
        wvec = wvec.astype(jnp.bfloat16).astype(jnp.float32)
        logits = jnp.sum(x16 * wvec, axis=2) + bia[0, col]
        lab = lab_ref[...]
        w = jnp.where(lab >= 0, 1.0, 0.0)
        elem = _bce_elem(logits, lab.astype(jnp.float32))
        loss_ref[...] = jnp.sum(w * elem).reshape(1, 1) / float(B * L)
        p = jax.nn.sigmoid(logits) * w
        return p, jax.lax.bitcast_convert_type(p, jnp.int32)

    pias, pbias = ia_pred(0, labias_ref, loss_ias_ref)
    piae, pbiae = ia_pred(1, labiae_ref, loss_iae_ref)

    def radix_select_scalar(pb2, kb):
        lo = jnp.int32(0)
        rng = 1 << 30
        while rng > 1:
            step = rng >> 3
            nsub = jnp.int32(0)
            for j in range(1, 8):
                ge = jnp.where(pb2 >= lo + jnp.int32(j * step), 1, 0)
                c = jnp.sum(jnp.sum(ge, axis=0))
                nsub = nsub + jnp.where(c >= kb, 1, 0)
            lo = lo + nsub * jnp.int32(step)
            rng = step
        return lo

    def count_ia(pb, thr):
        return jnp.sum(jnp.where(pb >= thr, 1, 0), axis=1, keepdims=True)

    def radix_select_ia(pb):
        lo = jnp.zeros((B, 1), jnp.int32)
        rng = 1 << 30
        while rng > 1:
            step = rng >> 3
            nsub = jnp.zeros((B, 1), jnp.int32)
            for j in range(1, 8):
                c = count_ia(pb, lo + jnp.int32(j * step))
                nsub = nsub + jnp.where(c >= k, 1, 0)
            lo = lo + nsub * jnp.int32(step)
            rng = step
        return lo

    thr_ias = jax.lax.bitcast_convert_type(radix_select_ia(pbias),
                                           jnp.float32)
    thr_iae = jax.lax.bitcast_convert_type(radix_select_ia(pbiae),
                                           jnp.float32)

    def table_mask(pb, m_ref):
        thr_bits = [radix_select_scalar(pb[b], k[b, 0]) for b in range(B)]
        thr = [jax.lax.bitcast_convert_type(t, jnp.float32)
               for t in thr_bits]
        strict = thr[0] == 0.0
        for b in range(B):
            p2 = jax.lax.bitcast_convert_type(pb[b], jnp.float32)
            gt = jnp.where(p2 > thr[b], 1.0, 0.0)
            ge = jnp.where(p2 >= thr[b], 1.0, 0.0)
            m_ref[pl.ds(b * L, L), :] = jnp.where(strict, gt, ge)

    table_mask(pbs, ms_ref)
    table_mask(pbe, me_ref)

    def ia_mask(p, thr2, m_ref):
        p2 = p[None, :, :]
        thr3 = thr2[:, :, None]
        strict = (thr3[0:1] == 0.0)
        gt = jnp.where(p2 > thr3, 1.0, 0.0)
        ge = jnp.where(p2 >= thr3, 1.0, 0.0)
        m_ref[...] = jnp.where(strict, gt, ge)

    ia_mask(pias, thr_ias, mias_ref)
    ia_mask(piae, thr_iae, miae_ref)


@functools.partial(jax.jit, static_argnames=())
def _run(table, attention_mask, table_labels_S, table_labels_E,
         table_labels_iaS, table_labels_iaE, ia_seq,
         W_S, b_S, W_E, b_E, W_iaS, b_iaS, W_iaE, b_iaE):
    t3 = table.reshape(B * L, L, D)
    wc = jnp.concatenate([W_S, W_E], axis=1)
    bc = jnp.concatenate([b_S, b_E]).reshape(1, 2)

    logits_S, logits_E = pl.pallas_call(
        _proj_body,
        grid=(NBLK,),
        in_specs=[
            pl.BlockSpec((BLK, L, D), lambda g: (g, 0, 0)),
            pl.BlockSpec((D, 2), lambda g: (0, 0)),
            pl.BlockSpec((1, 2), lambda g: (0, 0)),
        ],
        out_specs=[
            pl.BlockSpec((BLK, L), lambda g: (g, 0)),
            pl.BlockSpec((BLK, L), lambda g: (g, 0)),
        ],
        out_shape=[
            jax.ShapeDtypeStruct((B * L, L), jnp.float32),
            jax.ShapeDtypeStruct((B * L, L), jnp.float32),
        ],
    )(t3, wc, bc)

    wia = jnp.concatenate([W_iaS[:, 0], W_iaE[:, 0]]).reshape(1, 2 * D)
    bia = jnp.concatenate([b_iaS, b_iaE]).reshape(1, 2)

    outs = pl.pallas_call(
        _head_body,
        out_shape=[
            jax.ShapeDtypeStruct((1, 1), jnp.float32),
            jax.ShapeDtypeStruct((1, 1), jnp.float32),
            jax.ShapeDtypeStruct((1, 1), jnp.float32),
            jax.ShapeDtypeStruct((1, 1), jnp.float32),
            jax.ShapeDtypeStruct((B * L, L), jnp.float32),
            jax.ShapeDtypeStruct((B * L, L), jnp.float32),
            jax.ShapeDtypeStruct((B, B, L), jnp.float32),
            jax.ShapeDtypeStruct((B, B, L), jnp.float32),
        ],
    )(logits_S, logits_E,
      table_labels_S.reshape(B * L, L), table_labels_E.reshape(B * L, L),
      ia_seq, table_labels_iaS, table_labels_iaE, attention_mask, wia, bia)

    loss_S, loss_E, loss_iaS, loss_iaE, mS, mE, miaS, miaE = outs
    return (loss_S[0, 0], loss_E[0, 0], loss_iaS[0, 0], loss_iaE[0, 0],
            mS.reshape(B, L, L).astype(jnp.bool_),
            mE.reshape(B, L, L).astype(jnp.bool_),
            miaS.astype(jnp.bool_), miaE.astype(jnp.bool_))


def kernel(table, attention_mask, table_labels_S, table_labels_E,
           table_labels_iaS, table_labels_iaE, ia_seq,
           W_S, b_S, W_E, b_E, W_iaS, b_iaS, W_iaE, b_iaE):
    return _run(table, attention_mask, table_labels_S, table_labels_E,
                table_labels_iaS, table_labels_iaE, ia_seq,
                W_S, b_S, W_E, b_E, W_iaS, b_iaS, W_iaE, b_iaE)


# speedup vs baseline: 1.1324x; 1.1324x over previous
import functools

import jax
import jax.numpy as jnp
from jax.experimental import pallas as pl
from jax.experimental.pallas import tpu as pltpu

B, L, D = 4, 128, 768
SPAN_PRUNING = 0.3
BLK = 32
NBLK = (B * L) // BLK
N_ITER = 31


def _proj_body(t_ref, w_ref, b_ref, s_ref, e_ref):
    x = t_ref[...]
    x2 = x.reshape(BLK * L, D)
    r = jnp.dot(x2, w_ref[...], preferred_element_type=jnp.float32)
    r = r + b_ref[...]
    s_ref[...] = r[:, 0].reshape(BLK, L)
    e_ref[...] = r[:, 1].reshape(BLK, L)


def _bce_elem(logits, targets):
    return (jnp.maximum(logits, 0.0) - logits * targets
            + jnp.log1p(jnp.exp(-jnp.abs(logits))))


def _head_body(ls_ref, le_ref, labs_ref, labe_ref, ia_ref, labias_ref,
               labiae_ref, am_ref, wia_ref, bia_ref,
               loss_s_ref, loss_e_ref, loss_ias_ref, loss_iae_ref,
               ms_ref, me_ref, mias_ref, miae_ref):
    am = am_ref[...]
    msum = jnp.sum(am, axis=1, keepdims=True)
    ml = msum - 3.0
    ln = (ml * SPAN_PRUNING).astype(jnp.int32)
    ln = jnp.maximum(ln, 10)
    maxl = (ml * ml).astype(jnp.int32)
    k = jnp.minimum(ln, maxl)

    def table_pred(l_ref, lab_ref, loss_ref):
        logits = l_ref[...]
        lab = lab_ref[...]
        w = jnp.where(lab >= 0, 1.0, 0.0)
        elem = _bce_elem(logits, lab.astype(jnp.float32))
        loss_ref[...] = jnp.sum(jnp.sum(w * elem, axis=0)).reshape(1, 1) \
            / float(B * L * L)
        p = jax.nn.sigmoid(logits) * w
        return jax.lax.bitcast_convert_type(p.reshape(B, L, L), jnp.int32)

    pbs = table_pred(ls_ref, labs_ref, loss_s_ref)
    pbe = table_pred(le_ref, labe_ref, loss_e_ref)

    x16 = ia_ref[...].astype(jnp.bfloat16).astype(jnp.float32)
    wia = wia_ref[...]
    bia = bia_ref[...]

    def ia_pred(col, lab_ref, loss_ref):
        wvec = wia[0, col * D:(col + 1) * D].reshape(1, 1, D)
        wvec = wvec.astype(jnp.bfloat16).astype(jnp.float32)
        logits = jnp.sum(x16 * wvec, axis=2) + bia[0, col]
        lab = lab_ref[...]
        w = jnp.where(lab >= 0, 1.0, 0.0)
        elem = _bce_elem(logits, lab.astype(jnp.float32))
        loss_ref[...] = jnp.sum(w * elem).reshape(1, 1) / float(B * L)
        p = jax.nn.sigmoid(logits) * w
        return p, jax.lax.bitcast_convert_type(p, jnp.int32)

    pias, pbias = ia_pred(0, labias_ref, loss_ias_ref)
    piae, pbiae = ia_pred(1, labiae_ref, loss_iae_ref)

    def count_tab(pb, thr):
        ge = jnp.where(pb >= thr[:, :, None], 1, 0)
        return jnp.sum(jnp.sum(ge, axis=1), axis=1, keepdims=True)

    def count_ia(pb, thr):
        return jnp.sum(jnp.where(pb >= thr, 1, 0), axis=1, keepdims=True)

    def radix_select(pb, count_fn):
        lo = jnp.zeros((B, 1), jnp.int32)
        rng = 1 << 30
        while rng > 1:
            step = rng >> 2
            nsub = jnp.zeros((B, 1), jnp.int32)
            for j in range(1, 4):
                c = count_fn(pb, lo + jnp.int32(j * step))
                nsub = nsub + jnp.where(c >= k, 1, 0)
            lo = lo + nsub * jnp.int32(step)
            rng = step
        return lo

    def thr_of(pb, count_fn):
        bits = radix_select(pb, count_fn)
        return jax.lax.bitcast_convert_type(bits, jnp.float32)

    thr_s = thr_of(pbs, count_tab)
    thr_e = thr_of(pbe, count_tab)
    thr_ias = thr_of(pbias, count_ia)
    thr_iae = thr_of(pbiae, count_ia)

    def table_mask(pb, thr2, m_ref):
        p3 = jax.lax.bitcast_convert_type(pb, jnp.float32)
        thr = thr2[:, :, None]
        strict = (thr[0:1] == 0.0)
        gt = jnp.where(p3 > thr, 1.0, 0.0)
        ge = jnp.where(p3 >= thr, 1.0, 0.0)
        m_ref[...] = jnp.where(strict, gt, ge).reshape(B * L, L)

    table_mask(pbs, thr_s, ms_ref)
    table_mask(pbe, thr_e, me_ref)

    def ia_mask(p, thr2, m_ref):
        p2 = p[None, :, :]
        thr3 = thr2[:, :, None]
        strict = (thr3[0:1] == 0.0)
        gt = jnp.where(p2 > thr3, 1.0, 0.0)
        ge = jnp.where(p2 >= thr3, 1.0, 0.0)
        m_ref[...] = jnp.where(strict, gt, ge)

    ia_mask(pias, thr_ias, mias_ref)
    ia_mask(piae, thr_iae, miae_ref)


@functools.partial(jax.jit, static_argnames=())
def _run(table, attention_mask, table_labels_S, table_labels_E,
         table_labels_iaS, table_labels_iaE, ia_seq,
         W_S, b_S, W_E, b_E, W_iaS, b_iaS, W_iaE, b_iaE):
    t3 = table.reshape(B * L, L, D)
    wc = jnp.concatenate([W_S, W_E], axis=1)
    bc = jnp.concatenate([b_S, b_E]).reshape(1, 2)

    logits_S, logits_E = pl.pallas_call(
        _proj_body,
        grid=(NBLK,),
        in_specs=[
            pl.BlockSpec((BLK, L, D), lambda g: (g, 0, 0)),
            pl.BlockSpec((D, 2), lambda g: (0, 0)),
            pl.BlockSpec((1, 2), lambda g: (0, 0)),
        ],
        out_specs=[
            pl.BlockSpec((BLK, L), lambda g: (g, 0)),
            pl.BlockSpec((BLK, L), lambda g: (g, 0)),
        ],
        out_shape=[
            jax.ShapeDtypeStruct((B * L, L), jnp.float32),
            jax.ShapeDtypeStruct((B * L, L), jnp.float32),
        ],
    )(t3, wc, bc)

    wia = jnp.concatenate([W_iaS[:, 0], W_iaE[:, 0]]).reshape(1, 2 * D)
    bia = jnp.concatenate([b_iaS, b_iaE]).reshape(1, 2)

    outs = pl.pallas_call(
        _head_body,
        out_shape=[
            jax.ShapeDtypeStruct((1, 1), jnp.float32),
            jax.ShapeDtypeStruct((1, 1), jnp.float32),
            jax.ShapeDtypeStruct((1, 1), jnp.float32),
            jax.ShapeDtypeStruct((1, 1), jnp.float32),
            jax.ShapeDtypeStruct((B * L, L), jnp.float32),
            jax.ShapeDtypeStruct((B * L, L), jnp.float32),
            jax.ShapeDtypeStruct((B, B, L), jnp.float32),
            jax.ShapeDtypeStruct((B, B, L), jnp.float32),
        ],
    )(logits_S, logits_E,
      table_labels_S.reshape(B * L, L), table_labels_E.reshape(B * L, L),
      ia_seq, table_labels_iaS, table_labels_iaE, attention_mask, wia, bia)

    loss_S, loss_E, loss_iaS, loss_iaE, mS, mE, miaS, miaE = outs
    return (loss_S[0, 0], loss_E[0, 0], loss_iaS[0, 0], loss_iaE[0, 0],
            mS.reshape(B, L, L).astype(jnp.bool_),
            mE.reshape(B, L, L).astype(jnp.bool_),
            miaS.astype(jnp.bool_), miaE.astype(jnp.bool_))


def kernel(table, attention_mask, table_labels_S, table_labels_E,
           table_labels_iaS, table_labels_iaE, ia_seq,
           W_S, b_S, W_E, b_E, W_iaS, b_iaS, W_iaE, b_iaE):
    return _run(table, attention_mask, table_labels_S, table_labels_E,
                table_labels_iaS, table_labels_iaE, ia_seq,
                W_S, b_S, W_E, b_E, W_iaS, b_iaS, W_iaE, b_iaE)


# speedup vs baseline: 1.2064x; 1.0654x over previous
import functools

import jax
import jax.numpy as jnp
from jax.experimental import pallas as pl
from jax.experimental.pallas import tpu as pltpu

B, L, D = 4, 128, 768
SPAN_PRUNING = 0.3
BLK = 32
NBLK = (B * L) // BLK
N_ITER = 31


def _proj_body(t_ref, w_ref, b_ref, s_ref, e_ref):
    x = t_ref[...]
    x2 = x.reshape(BLK * L, D)
    r = jnp.dot(x2, w_ref[...], preferred_element_type=jnp.float32)
    r = r + b_ref[...]
    s_ref[...] = r[:, 0].reshape(BLK, L)
    e_ref[...] = r[:, 1].reshape(BLK, L)


def _bce_elem(logits, targets):
    return (jnp.maximum(logits, 0.0) - logits * targets
            + jnp.log1p(jnp.exp(-jnp.abs(logits))))


def _head_body(ls_ref, le_ref, labs_ref, labe_ref, ia_ref, labias_ref,
               labiae_ref, am_ref, wia_ref, bia_ref,
               loss_s_ref, loss_e_ref, loss_ias_ref, loss_iae_ref,
               ms_ref, me_ref, mias_ref, miae_ref):
    am = am_ref[...]
    msum = jnp.sum(am, axis=1, keepdims=True)
    ml = msum - 3.0
    ln = (ml * SPAN_PRUNING).astype(jnp.int32)
    ln = jnp.maximum(ln, 10)
    maxl = (ml * ml).astype(jnp.int32)
    k = jnp.minimum(ln, maxl)

    def table_pred(l_ref, lab_ref, loss_ref):
        logits = l_ref[...]
        lab = lab_ref[...]
        w = jnp.where(lab >= 0, 1.0, 0.0)
        elem = _bce_elem(logits, lab.astype(jnp.float32))
        loss_ref[...] = jnp.sum(jnp.sum(w * elem, axis=0)).reshape(1, 1) \
            / float(B * L * L)
        p = jax.nn.sigmoid(logits) * w
        return jax.lax.bitcast_convert_type(p.reshape(B, L, L), jnp.int32)

    pbs = table_pred(ls_ref, labs_ref, loss_s_ref)
    pbe = table_pred(le_ref, labe_ref, loss_e_ref)

    x16 = ia_ref[...].astype(jnp.bfloat16).astype(jnp.float32)
    wia = wia_ref[...]
    bia = bia_ref[...]

    def ia_pred(col, lab_ref, loss_ref):
        wvec = wia[0, col * D:(col + 1) * D].reshape(1, 1, D)
        wvec = wvec.astype(jnp.bfloat16).astype(jnp.float32)
        logits = jnp.sum(x16 * wvec, axis=2) + bia[0, col]
        lab = lab_ref[...]
        w = jnp.where(lab >= 0, 1.0, 0.0)
        elem = _bce_elem(logits, lab.astype(jnp.float32))
        loss_ref[...] = jnp.sum(w * elem).reshape(1, 1) / float(B * L)
        p = jax.nn.sigmoid(logits) * w
        return p, jax.lax.bitcast_convert_type(p, jnp.int32)

    pias, pbias = ia_pred(0, labias_ref, loss_ias_ref)
    piae, pbiae = ia_pred(1, labiae_ref, loss_iae_ref)

    def count_tab(pb, thr):
        ge = jnp.where(pb >= thr[:, :, None], 1, 0)
        return jnp.sum(jnp.sum(ge, axis=1), axis=1, keepdims=True)

    def count_ia(pb, thr):
        return jnp.sum(jnp.where(pb >= thr, 1, 0), axis=1, keepdims=True)

    def radix_select(pb, count_fn):
        lo = jnp.zeros((B, 1), jnp.int32)
        rng = 1 << 30
        while rng > 1:
            step = rng >> 1
            nsub = jnp.zeros((B, 1), jnp.int32)
            for j in range(1, 2):
                c = count_fn(pb, lo + jnp.int32(j * step))
                nsub = nsub + jnp.where(c >= k, 1, 0)
            lo = lo + nsub * jnp.int32(step)
            rng = step
        return lo

    def thr_of(pb, count_fn):
        bits = radix_select(pb, count_fn)
        return jax.lax.bitcast_convert_type(bits, jnp.float32)

    thr_s = thr_of(pbs, count_tab)
    thr_e = thr_of(pbe, count_tab)
    thr_ias = thr_of(pbias, count_ia)
    thr_iae = thr_of(pbiae, count_ia)

    def table_mask(pb, thr2, m_ref):
        p3 = jax.lax.bitcast_convert_type(pb, jnp.float32)
        thr = thr2[:, :, None]
        strict = (thr[0:1] == 0.0)
        gt = jnp.where(p3 > thr, 1.0, 0.0)
        ge = jnp.where(p3 >= thr, 1.0, 0.0)
        m_ref[...] = jnp.where(strict, gt, ge).reshape(B * L, L)

    table_mask(pbs, thr_s, ms_ref)
    table_mask(pbe, thr_e, me_ref)

    def ia_mask(p, thr2, m_ref):
        p2 = p[None, :, :]
        thr3 = thr2[:, :, None]
        strict = (thr3[0:1] == 0.0)
        gt = jnp.where(p2 > thr3, 1.0, 0.0)
        ge = jnp.where(p2 >= thr3, 1.0, 0.0)
        m_ref[...] = jnp.where(strict, gt, ge)

    ia_mask(pias, thr_ias, mias_ref)
    ia_mask(piae, thr_iae, miae_ref)


@functools.partial(jax.jit, static_argnames=())
def _run(table, attention_mask, table_labels_S, table_labels_E,
         table_labels_iaS, table_labels_iaE, ia_seq,
         W_S, b_S, W_E, b_E, W_iaS, b_iaS, W_iaE, b_iaE):
    t3 = table.reshape(B * L, L, D)
    wc = jnp.concatenate([W_S, W_E], axis=1)
    bc = jnp.concatenate([b_S, b_E]).reshape(1, 2)

    logits_S, logits_E = pl.pallas_call(
        _proj_body,
        grid=(NBLK,),
        in_specs=[
            pl.BlockSpec((BLK, L, D), lambda g: (g, 0, 0)),
            pl.BlockSpec((D, 2), lambda g: (0, 0)),
            pl.BlockSpec((1, 2), lambda g: (0, 0)),
        ],
        out_specs=[
            pl.BlockSpec((BLK, L), lambda g: (g, 0)),
            pl.BlockSpec((BLK, L), lambda g: (g, 0)),
        ],
        out_shape=[
            jax.ShapeDtypeStruct((B * L, L), jnp.float32),
            jax.ShapeDtypeStruct((B * L, L), jnp.float32),
        ],
    )(t3, wc, bc)

    wia = jnp.concatenate([W_iaS[:, 0], W_iaE[:, 0]]).reshape(1, 2 * D)
    bia = jnp.concatenate([b_iaS, b_iaE]).reshape(1, 2)

    outs = pl.pallas_call(
        _head_body,
        out_shape=[
            jax.ShapeDtypeStruct((1, 1), jnp.float32),
            jax.ShapeDtypeStruct((1, 1), jnp.float32),
            jax.ShapeDtypeStruct((1, 1), jnp.float32),
            jax.ShapeDtypeStruct((1, 1), jnp.float32),
            jax.ShapeDtypeStruct((B * L, L), jnp.float32),
            jax.ShapeDtypeStruct((B * L, L), jnp.float32),
            jax.ShapeDtypeStruct((B, B, L), jnp.float32),
            jax.ShapeDtypeStruct((B, B, L), jnp.float32),
        ],
    )(logits_S, logits_E,
      table_labels_S.reshape(B * L, L), table_labels_E.reshape(B * L, L),
      ia_seq, table_labels_iaS, table_labels_iaE, attention_mask, wia, bia)

    loss_S, loss_E, loss_iaS, loss_iaE, mS, mE, miaS, miaE = outs
    return (loss_S[0, 0], loss_E[0, 0], loss_iaS[0, 0], loss_iaE[0, 0],
            mS.reshape(B, L, L).astype(jnp.bool_),
            mE.reshape(B, L, L).astype(jnp.bool_),
            miaS.astype(jnp.bool_), miaE.astype(jnp.bool_))


def kernel(table, attention_mask, table_labels_S, table_labels_E,
           table_labels_iaS, table_labels_iaE, ia_seq,
           W_S, b_S, W_E, b_E, W_iaS, b_iaS, W_iaE, b_iaE):
    return _run(table, attention_mask, table_labels_S, table_labels_E,
                table_labels_iaS, table_labels_iaE, ia_seq,
                W_S, b_S, W_E, b_E, W_iaS, b_iaS, W_iaE, b_iaE)


# speedup vs baseline: 1.2118x; 1.0045x over previous
import functools

import jax
import jax.numpy as jnp
from jax.experimental import pallas as pl
from jax.experimental.pallas import tpu as pltpu

B, L, D = 4, 128, 768
SPAN_PRUNING = 0.3
BLK = 32
NBLK = (B * L) // BLK


def _proj_body(t_ref, w_ref, b_ref, s_ref, e_ref):
    x = t_ref[...]
    x2 = x.reshape(BLK * L, D)
    r = jnp.dot(x2, w_ref[...], preferred_element_type=jnp.float32)
    r = r + b_ref[...]
    s_ref[...] = r[:, 0].reshape(BLK, L)
    e_ref[...] = r[:, 1].reshape(BLK, L)


def _bce_elem(logits, targets):
    return (jnp.maximum(logits, 0.0) - logits * targets
            + jnp.log1p(jnp.exp(-jnp.abs(logits))))


def _head_body(ls_ref, le_ref, labs_ref, labe_ref, ia_ref, labias_ref,
               labiae_ref, am_ref, wia_ref, bia_ref,
               loss_s_ref, loss_e_ref, loss_ias_ref, loss_iae_ref,
               ms_ref, me_ref, mias_ref, miae_ref):
    am = am_ref[...]
    msum = jnp.sum(am, axis=1, keepdims=True)
    ml = msum - 3.0
    ln = (ml * SPAN_PRUNING).astype(jnp.int32)
    ln = jnp.maximum(ln, 10)
    maxl = (ml * ml).astype(jnp.int32)
    k = jnp.minimum(ln, maxl)

    def table_pred(l_ref, lab_ref, loss_ref):
        logits = l_ref[...]
        lab = lab_ref[...]
        w = jnp.where(lab >= 0, 1.0, 0.0)
        elem = _bce_elem(logits, lab.astype(jnp.float32))
        loss_ref[...] = jnp.sum(jnp.sum(w * elem, axis=0)).reshape(1, 1) \
            / float(B * L * L)
        p = jax.nn.sigmoid(logits) * w
        return jax.lax.bitcast_convert_type(p.reshape(B, L, L), jnp.int32)

    pbs = table_pred(ls_ref, labs_ref, loss_s_ref)
    pbe = table_pred(le_ref, labe_ref, loss_e_ref)

    x16 = ia_ref[...].astype(jnp.bfloat16).astype(jnp.float32)
    wia = wia_ref[...]
    bia = bia_ref[...]

    def ia_pred(col, lab_ref, loss_ref):
        wvec = wia[0, col * D:(col + 1) * D].reshape(1, 1, D)
        wvec = wvec.astype(jnp.bfloat16).astype(jnp.float32)
        logits = jnp.sum(x16 * wvec, axis=2) + bia[0, col]
        lab = lab_ref[...]
        w = jnp.where(lab >= 0, 1.0, 0.0)
        elem = _bce_elem(logits, lab.astype(jnp.float32))
        loss_ref[...] = jnp.sum(w * elem).reshape(1, 1) / float(B * L)
        p = jax.nn.sigmoid(logits) * w
        return p, jax.lax.bitcast_convert_type(p, jnp.int32)

    pias, pbias = ia_pred(0, labias_ref, loss_ias_ref)
    piae, pbiae = ia_pred(1, labiae_ref, loss_iae_ref)

    def count_tab(pb, thr):
        ge = jnp.where(pb >= thr[:, :, None], 1, 0)
        return jnp.sum(jnp.sum(ge, axis=1), axis=1, keepdims=True)

    def count_ia(pb, thr):
        return jnp.sum(jnp.where(pb >= thr, 1, 0), axis=1, keepdims=True)

    def radix_select(pb, count_fn):
        lo = jnp.zeros((B, 1), jnp.int32)
        rng = 1 << 30
        while rng > 1:
            step = rng >> 1
            nsub = jnp.zeros((B, 1), jnp.int32)
            for j in range(1, 2):
                c = count_fn(pb, lo + jnp.int32(j * step))
                nsub = nsub + jnp.where(c >= k, 1, 0)
            lo = lo + nsub * jnp.int32(step)
            rng = step
        return lo

    def thr_of(pb, count_fn):
        bits = radix_select(pb, count_fn)
        return jax.lax.bitcast_convert_type(bits, jnp.float32)

    thr_s = thr_of(pbs, count_tab)
    thr_e = thr_of(pbe, count_tab)
    thr_ias = thr_of(pbias, count_ia)
    thr_iae = thr_of(pbiae, count_ia)

    def table_mask(pb, thr2, m_ref):
        p3 = jax.lax.bitcast_convert_type(pb, jnp.float32)
        thr = thr2[:, :, None]
        strict = (thr[0:1] == 0.0)
        gt = jnp.where(p3 > thr, 1.0, 0.0)
        ge = jnp.where(p3 >= thr, 1.0, 0.0)
        m_ref[...] = jnp.where(strict, gt, ge).reshape(B * L, L)

    table_mask(pbs, thr_s, ms_ref)
    table_mask(pbe, thr_e, me_ref)

    def ia_mask(p, thr2, m_ref):
        p2 = p[None, :, :]
        thr3 = thr2[:, :, None]
        strict = (thr3[0:1] == 0.0)
        gt = jnp.where(p2 > thr3, 1.0, 0.0)
        ge = jnp.where(p2 >= thr3, 1.0, 0.0)
        m_ref[...] = jnp.where(strict, gt, ge)

    ia_mask(pias, thr_ias, mias_ref)
    ia_mask(piae, thr_iae, miae_ref)


@functools.partial(jax.jit, static_argnames=())
def _run(table, attention_mask, table_labels_S, table_labels_E,
         table_labels_iaS, table_labels_iaE, ia_seq,
         W_S, b_S, W_E, b_E, W_iaS, b_iaS, W_iaE, b_iaE):
    t3 = table.reshape(B * L, L, D)
    wc = jnp.concatenate([W_S, W_E], axis=1)
    bc = jnp.concatenate([b_S, b_E]).reshape(1, 2)

    logits_S, logits_E = pl.pallas_call(
        _proj_body,
        grid=(NBLK,),
        in_specs=[
            pl.BlockSpec((BLK, L, D), lambda g: (g, 0, 0)),
            pl.BlockSpec((D, 2), lambda g: (0, 0)),
            pl.BlockSpec((1, 2), lambda g: (0, 0)),
        ],
        out_specs=[
            pl.BlockSpec((BLK, L), lambda g: (g, 0)),
            pl.BlockSpec((BLK, L), lambda g: (g, 0)),
        ],
        out_shape=[
            jax.ShapeDtypeStruct((B * L, L), jnp.float32),
            jax.ShapeDtypeStruct((B * L, L), jnp.float32),
        ],
    )(t3, wc, bc)

    wia = jnp.concatenate([W_iaS[:, 0], W_iaE[:, 0]]).reshape(1, 2 * D)
    bia = jnp.concatenate([b_iaS, b_iaE]).reshape(1, 2)

    outs = pl.pallas_call(
        _head_body,
        out_shape=[
            jax.ShapeDtypeStruct((1, 1), jnp.float32),
            jax.ShapeDtypeStruct((1, 1), jnp.float32),
            jax.ShapeDtypeStruct((1, 1), jnp.float32),
            jax.ShapeDtypeStruct((1, 1), jnp.float32),
            jax.ShapeDtypeStruct((B * L, L), jnp.float32),
            jax.ShapeDtypeStruct((B * L, L), jnp.float32),
            jax.ShapeDtypeStruct((B, B, L), jnp.float32),
            jax.ShapeDtypeStruct((B, B, L), jnp.float32),
        ],
    )(logits_S, logits_E,
      table_labels_S.reshape(B * L, L), table_labels_E.reshape(B * L, L),
      ia_seq, table_labels_iaS, table_labels_iaE, attention_mask, wia, bia)

    loss_S, loss_E, loss_iaS, loss_iaE, mS, mE, miaS, miaE = outs
    return (loss_S[0, 0], loss_E[0, 0], loss_iaS[0, 0], loss_iaE[0, 0],
            mS.reshape(B, L, L).astype(jnp.bool_),
            mE.reshape(B, L, L).astype(jnp.bool_),
            miaS.astype(jnp.bool_), miaE.astype(jnp.bool_))


def kernel(table, attention_mask, table_labels_S, table_labels_E,
           table_labels_iaS, table_labels_iaE, ia_seq,
           W_S, b_S, W_E, b_E, W_iaS, b_iaS, W_iaE, b_iaE):
    return _run(table, attention_mask, table_labels_S, table_labels_E,
                table_labels_iaS, table_labels_iaE, ia_seq,
                W_S, b_S, W_E, b_E, W_iaS, b_iaS, W_iaE, b_iaE)
